# Initial kernel scaffold; baseline (speedup 1.0000x reference)
import jax
import jax.numpy as jnp
from jax.experimental import pallas as pl


def kernel(x, matbert_table, W, b):
    raise NotImplementedError("write your pallas kernel here")


# speedup vs baseline: 2.0806x; 2.0806x over previous
import functools

import numpy as np
import jax
import jax.numpy as jnp
from jax import lax
from jax.experimental import pallas as pl
from jax.experimental.pallas import tpu as pltpu
from jax.experimental.pallas import tpu_sc as plsc

VOCAB = 64
SEQ = 512
D = 128
H = 768

NUM_CORES = 2
NUM_SUBCORES = 16
NW = NUM_CORES * NUM_SUBCORES

LANES = 16


def _pe_flat():
    pos = np.arange(SEQ)[:, None].astype(np.float32)
    i = np.arange(D // 2)[None, :].astype(np.float32)
    ang = pos / np.power(10000.0, (2.0 * i) / float(D))
    pe = np.zeros((SEQ, D), dtype=np.float32)
    pe[:, 0::2] = np.sin(ang)
    pe[:, 1::2] = np.cos(ang)
    return pe.reshape(-1)


_PE_CONST = _pe_flat()


def _matmul_body(a_ref, w_ref, b_ref, o_ref):
    o_ref[...] = (
        jnp.dot(a_ref[...], w_ref[...], preferred_element_type=jnp.float32)
        + b_ref[...]
    )


def _reduced_table(matbert_table, W, b):
    return pl.pallas_call(
        _matmul_body,
        out_shape=jax.ShapeDtypeStruct((VOCAB, D), jnp.float32),
    )(matbert_table, W, b.reshape(1, D))


def _make_sc_kernel(batch):
    total_rows = batch * SEQ
    rows_per_w = total_rows // NW
    chunk = 128
    nchunk = rows_per_w // chunk

    mesh = plsc.VectorSubcoreMesh(
        core_axis_name="c",
        subcore_axis_name="s",
        num_cores=NUM_CORES,
        num_subcores=NUM_SUBCORES,
    )

    @functools.partial(
        pl.kernel,
        out_type=jax.ShapeDtypeStruct((total_rows * D,), jnp.float32),
        mesh=mesh,
        compiler_params=pltpu.CompilerParams(needs_layout_passes=False),
        scratch_types=[
            pltpu.VMEM((VOCAB * D,), jnp.float32),
            pltpu.VMEM((SEQ * D,), jnp.float32),
            pltpu.VMEM((rows_per_w,), jnp.int32),
            pltpu.VMEM((chunk * D,), jnp.float32),
        ],
    )
    def sc_gather(x_hbm, red_hbm, pe_hbm, out_hbm, tab_v, pe_v, tok_v, buf):
        wid = lax.axis_index("s") * NUM_CORES + lax.axis_index("c")
        base = wid * rows_per_w
        pltpu.sync_copy(red_hbm, tab_v)
        pltpu.sync_copy(pe_hbm, pe_v)
        pltpu.sync_copy(x_hbm.at[pl.ds(base, rows_per_w)], tok_v)

        iota = lax.iota(jnp.int32, 16)

        def chunk_body(k, _):
            l0 = lax.rem(k * chunk, SEQ)
            peoff = l0 * D

            def row_body(r, _):
                tok = plsc.load_gather(
                    tok_v, [jnp.full((16,), k * chunk, jnp.int32) + r]
                )
                rowbase = tok * D + iota
                for c in range(D // LANES):
                    vals = plsc.load_gather(tab_v, [rowbase + (c * LANES)])
                    pe = pe_v[pl.ds(peoff + r * D + c * LANES, 16)]
                    buf[pl.ds(r * D + c * LANES, 16)] = vals + pe
                return 0

            lax.fori_loop(0, chunk, row_body, 0)
            pltpu.sync_copy(
                buf, out_hbm.at[pl.ds((base + k * chunk) * D, chunk * D)]
            )
            return 0

        lax.fori_loop(0, nchunk, chunk_body, 0)

    return sc_gather


def kernel(x, matbert_table, W, b):
    batch, seq = x.shape
    reduced = _reduced_table(matbert_table, W, b)
    pe = jnp.asarray(_PE_CONST)
    sc = _make_sc_kernel(batch)
    out_flat = sc(x.reshape(-1), reduced.reshape(-1), pe)
    return out_flat.reshape(batch, seq, D)


# speedup vs baseline: 2.2923x; 1.1018x over previous
import functools

import numpy as np
import jax
import jax.numpy as jnp
from jax import lax
from jax.experimental import pallas as pl
from jax.experimental.pallas import tpu as pltpu
from jax.experimental.pallas import tpu_sc as plsc

VOCAB = 64
SEQ = 512
D = 128
H = 768

NUM_CORES = 2
NUM_SUBCORES = 16
NW = NUM_CORES * NUM_SUBCORES

LANES = 16


def _pe_flat():
    pos = np.arange(SEQ)[:, None].astype(np.float32)
    i = np.arange(D // 2)[None, :].astype(np.float32)
    ang = pos / np.power(10000.0, (2.0 * i) / float(D))
    pe = np.zeros((SEQ, D), dtype=np.float32)
    pe[:, 0::2] = np.sin(ang)
    pe[:, 1::2] = np.cos(ang)
    return pe.reshape(-1)


_PE_CONST = _pe_flat()


def _matmul_body(a_ref, w_ref, b_ref, o_ref):
    o_ref[...] = (
        jnp.dot(a_ref[...], w_ref[...], preferred_element_type=jnp.float32)
        + b_ref[...]
    )


def _reduced_table(matbert_table, W, b):
    return pl.pallas_call(
        _matmul_body,
        out_shape=jax.ShapeDtypeStruct((VOCAB, D), jnp.float32),
    )(matbert_table, W, b.reshape(1, D))


def _make_sc_kernel(batch):
    total_rows = batch * SEQ
    rows_per_w = total_rows // NW
    chunk = 128
    nchunk = rows_per_w // chunk

    mesh = plsc.VectorSubcoreMesh(
        core_axis_name="c",
        subcore_axis_name="s",
        num_cores=NUM_CORES,
        num_subcores=NUM_SUBCORES,
    )

    @functools.partial(
        pl.kernel,
        out_type=jax.ShapeDtypeStruct((total_rows * D,), jnp.float32),
        mesh=mesh,
        compiler_params=pltpu.CompilerParams(needs_layout_passes=False),
        scratch_types=[
            pltpu.VMEM((VOCAB * D,), jnp.float32),
            pltpu.VMEM((SEQ * D,), jnp.float32),
            pltpu.VMEM((rows_per_w,), jnp.int32),
            pltpu.VMEM((chunk * D,), jnp.float32),
            pltpu.VMEM((chunk * D,), jnp.float32),
            pltpu.SemaphoreType.DMA,
            pltpu.SemaphoreType.DMA,
        ],
    )
    def sc_gather(
        x_hbm, red_hbm, pe_hbm, out_hbm, tab_v, pe_v, tok_v, buf0, buf1,
        sem0, sem1,
    ):
        wid = lax.axis_index("s") * NUM_CORES + lax.axis_index("c")
        base = wid * rows_per_w
        pltpu.sync_copy(red_hbm, tab_v)
        pltpu.sync_copy(pe_hbm, pe_v)
        pltpu.sync_copy(x_hbm.at[pl.ds(base, rows_per_w)], tok_v)

        iota = lax.iota(jnp.int32, 16)
        unroll = 16

        def compute_chunk(k, buf):
            peoff = lax.rem(k * chunk, SEQ) * D

            def blk(i, _):
                r0 = i * unroll
                for rr in range(unroll):
                    r = r0 + rr
                    tok = plsc.load_gather(
                        tok_v, [jnp.full((16,), k * chunk + r, jnp.int32)]
                    )
                    rowbase = tok * D + iota
                    for c in range(D // LANES):
                        vals = plsc.load_gather(tab_v, [rowbase + (c * LANES)])
                        pe = pe_v[pl.ds(peoff + r * D + c * LANES, 16)]
                        buf[pl.ds(r * D + c * LANES, 16)] = vals + pe
                return 0

            lax.fori_loop(0, chunk // unroll, blk, 0)

        def out_slice(k):
            return out_hbm.at[pl.ds((base + k * chunk) * D, chunk * D)]

        compute_chunk(0, buf0)
        pltpu.async_copy(buf0, out_slice(0), sem0)
        compute_chunk(1, buf1)
        pltpu.async_copy(buf1, out_slice(1), sem1)

        def pair(kk, _):
            k0 = kk * 2
            k1 = k0 + 1
            pltpu.make_async_copy(buf0, out_slice(k0 - 2), sem0).wait()
            compute_chunk(k0, buf0)
            pltpu.async_copy(buf0, out_slice(k0), sem0)
            pltpu.make_async_copy(buf1, out_slice(k1 - 2), sem1).wait()
            compute_chunk(k1, buf1)
            pltpu.async_copy(buf1, out_slice(k1), sem1)
            return 0

        lax.fori_loop(1, nchunk // 2, pair, 0)
        pltpu.make_async_copy(buf0, out_slice(nchunk - 2), sem0).wait()
        pltpu.make_async_copy(buf1, out_slice(nchunk - 1), sem1).wait()

    return sc_gather


def kernel(x, matbert_table, W, b):
    batch, seq = x.shape
    reduced = _reduced_table(matbert_table, W, b)
    pe = jnp.asarray(_PE_CONST)
    sc = _make_sc_kernel(batch)
    out_flat = sc(x.reshape(-1), reduced.reshape(-1), pe)
    return out_flat.reshape(batch, seq, D)


# speedup vs baseline: 6.6986x; 2.9222x over previous
import functools

import numpy as np
import jax
import jax.numpy as jnp
from jax import lax
from jax.experimental import pallas as pl
from jax.experimental.pallas import tpu as pltpu
from jax.experimental.pallas import tpu_sc as plsc

VOCAB = 64
SEQ = 512
D = 128
H = 768

NUM_CORES = 2
NUM_SUBCORES = 16
NW = NUM_CORES * NUM_SUBCORES

LANES = 16


def _pe_flat():
    pos = np.arange(SEQ)[:, None].astype(np.float32)
    i = np.arange(D // 2)[None, :].astype(np.float32)
    ang = pos / np.power(10000.0, (2.0 * i) / float(D))
    pe = np.zeros((SEQ, D), dtype=np.float32)
    pe[:, 0::2] = np.sin(ang)
    pe[:, 1::2] = np.cos(ang)
    return pe.reshape(-1)


_PE_CONST = _pe_flat()


def _matmul_body(a_ref, w_ref, b_ref, o_ref):
    o_ref[...] = (
        jnp.dot(a_ref[...], w_ref[...], preferred_element_type=jnp.float32)
        + b_ref[...]
    )


def _reduced_table(matbert_table, W, b):
    return pl.pallas_call(
        _matmul_body,
        out_shape=jax.ShapeDtypeStruct((VOCAB, D), jnp.float32),
    )(matbert_table, W, b.reshape(1, D))


def _make_sc_kernel(batch):
    total_rows = batch * SEQ
    rows_per_w = total_rows // NW
    chunk = 128
    nchunk = rows_per_w // chunk

    mesh = plsc.VectorSubcoreMesh(
        core_axis_name="c",
        subcore_axis_name="s",
        num_cores=NUM_CORES,
        num_subcores=NUM_SUBCORES,
    )

    @functools.partial(
        pl.kernel,
        out_type=jax.ShapeDtypeStruct((total_rows * D,), jnp.float32),
        mesh=mesh,
        compiler_params=pltpu.CompilerParams(needs_layout_passes=False),
        scratch_types=[
            pltpu.VMEM((VOCAB * D,), jnp.float32),
            pltpu.VMEM((SEQ * D,), jnp.float32),
            pltpu.VMEM((rows_per_w,), jnp.int32),
            pltpu.VMEM((chunk * D,), jnp.float32),
            pltpu.VMEM((chunk * D,), jnp.float32),
            pltpu.SemaphoreType.DMA,
            pltpu.SemaphoreType.DMA,
        ],
    )
    def sc_gather(
        x_hbm, red_hbm, pe_hbm, out_hbm, tab_v, pe_v, tok_v, buf0, buf1,
        sem0, sem1,
    ):
        wid = lax.axis_index("s") * NUM_CORES + lax.axis_index("c")
        base = wid * rows_per_w
        pltpu.sync_copy(red_hbm, tab_v)
        pltpu.sync_copy(pe_hbm, pe_v)
        pltpu.sync_copy(x_hbm.at[pl.ds(base, rows_per_w)], tok_v)

        iota = lax.iota(jnp.int32, 16)
        unroll = 16

        ncol = D // LANES

        def compute_chunk(k, buf):
            peoff = lax.rem(k * chunk, SEQ) * D

            @plsc.parallel_loop(0, chunk, step=unroll)
            def _(r0):
                def row_loads(r):
                    tok = plsc.load_gather(
                        tok_v, [jnp.full((16,), k * chunk + r, jnp.int32)]
                    )
                    rowbase = tok * D + iota
                    gs = [
                        plsc.load_gather(tab_v, [rowbase + (c * LANES)])
                        for c in range(ncol)
                    ]
                    pes = [
                        pe_v[pl.ds(peoff + r * D + c * LANES, 16)]
                        for c in range(ncol)
                    ]
                    return gs, pes

                def row_store(r, staged):
                    gs, pes = staged
                    for c in range(ncol):
                        buf[pl.ds(r * D + c * LANES, 16)] = gs[c] + pes[c]

                staged = row_loads(r0)
                for rr in range(1, unroll):
                    nxt = row_loads(r0 + rr)
                    row_store(r0 + rr - 1, staged)
                    staged = nxt
                row_store(r0 + unroll - 1, staged)

        def out_slice(k):
            return out_hbm.at[pl.ds((base + k * chunk) * D, chunk * D)]

        compute_chunk(0, buf0)
        pltpu.async_copy(buf0, out_slice(0), sem0)
        compute_chunk(1, buf1)
        pltpu.async_copy(buf1, out_slice(1), sem1)

        def pair(kk, _):
            k0 = kk * 2
            k1 = k0 + 1
            pltpu.make_async_copy(buf0, out_slice(k0 - 2), sem0).wait()
            compute_chunk(k0, buf0)
            pltpu.async_copy(buf0, out_slice(k0), sem0)
            pltpu.make_async_copy(buf1, out_slice(k1 - 2), sem1).wait()
            compute_chunk(k1, buf1)
            pltpu.async_copy(buf1, out_slice(k1), sem1)
            return 0

        lax.fori_loop(1, nchunk // 2, pair, 0)
        pltpu.make_async_copy(buf0, out_slice(nchunk - 2), sem0).wait()
        pltpu.make_async_copy(buf1, out_slice(nchunk - 1), sem1).wait()

    return sc_gather


def kernel(x, matbert_table, W, b):
    batch, seq = x.shape
    reduced = _reduced_table(matbert_table, W, b)
    pe = jnp.asarray(_PE_CONST)
    sc = _make_sc_kernel(batch)
    out_flat = sc(x.reshape(-1), reduced.reshape(-1), pe)
    return out_flat.reshape(batch, seq, D)


# speedup vs baseline: 6.7943x; 1.0143x over previous
import functools

import numpy as np
import jax
import jax.numpy as jnp
from jax import lax
from jax.experimental import pallas as pl
from jax.experimental.pallas import tpu as pltpu
from jax.experimental.pallas import tpu_sc as plsc

VOCAB = 64
SEQ = 512
D = 128
H = 768

NUM_CORES = 2
NUM_SUBCORES = 16
NW = NUM_CORES * NUM_SUBCORES

LANES = 16
LPW = SEQ // NW


def _pe_flat():
    pos = np.arange(SEQ)[:, None].astype(np.float32)
    i = np.arange(D // 2)[None, :].astype(np.float32)
    ang = pos / np.power(10000.0, (2.0 * i) / float(D))
    pe = np.zeros((SEQ, D), dtype=np.float32)
    pe[:, 0::2] = np.sin(ang)
    pe[:, 1::2] = np.cos(ang)
    return pe.reshape(-1)


_PE_CONST = _pe_flat()


def _matmul_body(a_ref, w_ref, b_ref, o_ref):
    o_ref[...] = (
        jnp.dot(a_ref[...], w_ref[...], preferred_element_type=jnp.float32)
        + b_ref[...]
    )


def _reduced_table(matbert_table, W, b):
    return pl.pallas_call(
        _matmul_body,
        out_shape=jax.ShapeDtypeStruct((VOCAB, D), jnp.float32),
    )(matbert_table, W, b.reshape(1, D))


def _make_sc_kernel(batch):
    bblk = 32
    pblk = 8
    nbblk = batch // bblk
    npblk = LPW // pblk
    nchunk = npblk * nbblk
    ncol = D // LANES

    mesh = plsc.VectorSubcoreMesh(
        core_axis_name="c",
        subcore_axis_name="s",
        num_cores=NUM_CORES,
        num_subcores=NUM_SUBCORES,
    )

    @functools.partial(
        pl.kernel,
        out_type=jax.ShapeDtypeStruct((batch, SEQ, D), jnp.float32),
        mesh=mesh,
        compiler_params=pltpu.CompilerParams(needs_layout_passes=False),
        scratch_types=[
            pltpu.VMEM((VOCAB * D,), jnp.float32),
            pltpu.VMEM((LPW * D,), jnp.float32),
            pltpu.VMEM((LPW, batch), jnp.int32),
            pltpu.VMEM((bblk, pblk, D), jnp.float32),
            pltpu.VMEM((bblk, pblk, D), jnp.float32),
            pltpu.SemaphoreType.DMA,
            pltpu.SemaphoreType.DMA,
        ],
    )
    def sc_gather(
        xt_hbm, red_hbm, pe_hbm, out_hbm, tab_v, pe_v, tok_v, buf0, buf1,
        sem0, sem1,
    ):
        wid = lax.axis_index("s") * NUM_CORES + lax.axis_index("c")
        l0 = wid * LPW
        pltpu.sync_copy(red_hbm, tab_v)
        pltpu.sync_copy(pe_hbm.at[pl.ds(l0 * D, LPW * D)], pe_v)
        pltpu.sync_copy(xt_hbm.at[pl.ds(l0, LPW)], tok_v)

        iota = lax.iota(jnp.int32, 16)

        def compute_chunk(q, buf):
            sb = lax.div(q, nbblk)
            b0 = lax.rem(q, nbblk) * bblk

            def pos_body(j, _):
                dl = sb * pblk + j
                lvec = jnp.full((16,), dl, jnp.int32)
                pes = [
                    pe_v[pl.ds(dl * D + c * LANES, 16)] for c in range(ncol)
                ]

                @plsc.parallel_loop(0, bblk, step=16)
                def _(g0):
                    def row_loads(rr):
                        rb = plsc.load_gather(
                            tok_v,
                            [lvec, jnp.full((16,), b0 + g0 + rr, jnp.int32)],
                        ) * D + iota
                        return [
                            plsc.load_gather(tab_v, [rb + (c * LANES)])
                            for c in range(ncol)
                        ]

                    def row_store(rr, gs):
                        for c in range(ncol):
                            buf[g0 + rr, j, pl.ds(c * LANES, 16)] = (
                                gs[c] + pes[c]
                            )

                    staged = row_loads(0)
                    for rr in range(1, 16):
                        nxt = row_loads(rr)
                        row_store(rr - 1, staged)
                        staged = nxt
                    row_store(15, staged)

                return 0

            lax.fori_loop(0, pblk, pos_body, 0)

        def out_slice(q):
            sb = lax.div(q, nbblk)
            b0 = lax.rem(q, nbblk) * bblk
            return out_hbm.at[pl.ds(b0, bblk), pl.ds(l0 + sb * pblk, pblk)]

        compute_chunk(0, buf0)
        pltpu.async_copy(buf0, out_slice(0), sem0)
        compute_chunk(1, buf1)
        pltpu.async_copy(buf1, out_slice(1), sem1)

        def pair(kk, _):
            q0 = kk * 2
            q1 = q0 + 1
            pltpu.make_async_copy(buf0, out_slice(q0 - 2), sem0).wait()
            compute_chunk(q0, buf0)
            pltpu.async_copy(buf0, out_slice(q0), sem0)
            pltpu.make_async_copy(buf1, out_slice(q1 - 2), sem1).wait()
            compute_chunk(q1, buf1)
            pltpu.async_copy(buf1, out_slice(q1), sem1)
            return 0

        lax.fori_loop(1, nchunk // 2, pair, 0)
        pltpu.make_async_copy(buf0, out_slice(nchunk - 2), sem0).wait()
        pltpu.make_async_copy(buf1, out_slice(nchunk - 1), sem1).wait()

    return sc_gather


def kernel(x, matbert_table, W, b):
    batch, seq = x.shape
    reduced = _reduced_table(matbert_table, W, b)
    pe = jnp.asarray(_PE_CONST)
    sc = _make_sc_kernel(batch)
    return sc(x.T, reduced.reshape(-1), pe)


# speedup vs baseline: 10.2942x; 1.5151x over previous
import functools

import numpy as np
import jax
import jax.numpy as jnp
from jax import lax
from jax.experimental import pallas as pl
from jax.experimental.pallas import tpu as pltpu
from jax.experimental.pallas import tpu_sc as plsc

VOCAB = 64
SEQ = 512
D = 128
H = 768

NUM_CORES = 2
NUM_SUBCORES = 16
NW = NUM_CORES * NUM_SUBCORES

LANES = 16
LPW = SEQ // NW


def _pe_flat():
    pos = np.arange(SEQ)[:, None].astype(np.float32)
    i = np.arange(D // 2)[None, :].astype(np.float32)
    ang = pos / np.power(10000.0, (2.0 * i) / float(D))
    pe = np.zeros((SEQ, D), dtype=np.float32)
    pe[:, 0::2] = np.sin(ang)
    pe[:, 1::2] = np.cos(ang)
    return pe.reshape(-1)


_PE_CONST = _pe_flat()


def _matmul_body(a_ref, w_ref, b_ref, o_ref):
    o_ref[...] = (
        jnp.dot(a_ref[...], w_ref[...], preferred_element_type=jnp.float32)
        + b_ref[...]
    )


def _reduced_table(matbert_table, W, b):
    return pl.pallas_call(
        _matmul_body,
        out_shape=jax.ShapeDtypeStruct((VOCAB, D), jnp.float32),
    )(matbert_table, W, b.reshape(1, D))


def _make_sc_kernel(batch):
    bblk = 32
    pblk = 8
    nbblk = batch // bblk
    npblk = LPW // pblk
    nchunk = npblk * nbblk
    ncol = D // LANES

    mesh = plsc.VectorSubcoreMesh(
        core_axis_name="c",
        subcore_axis_name="s",
        num_cores=NUM_CORES,
        num_subcores=NUM_SUBCORES,
    )

    @functools.partial(
        pl.kernel,
        out_type=jax.ShapeDtypeStruct((batch, SEQ, D), jnp.float32),
        mesh=mesh,
        compiler_params=pltpu.CompilerParams(needs_layout_passes=False),
        scratch_types=[
            pltpu.VMEM((VOCAB * D,), jnp.float32),
            pltpu.VMEM((LPW * D,), jnp.float32),
            pltpu.VMEM((LPW, batch), jnp.int32),
            pltpu.VMEM((bblk, pblk, D), jnp.float32),
            pltpu.VMEM((bblk, pblk, D), jnp.float32),
            pltpu.SemaphoreType.DMA,
            pltpu.SemaphoreType.DMA,
        ],
    )
    def sc_gather(
        xt_hbm, red_hbm, pe_hbm, out_hbm, tab_v, pe_v, tok_v, buf0, buf1,
        sem0, sem1,
    ):
        wid = lax.axis_index("s") * NUM_CORES + lax.axis_index("c")
        l0 = wid * LPW
        pltpu.sync_copy(red_hbm, tab_v)
        pltpu.sync_copy(pe_hbm.at[pl.ds(l0 * D, LPW * D)], pe_v)
        pltpu.sync_copy(xt_hbm.at[pl.ds(l0, LPW)], tok_v)

        iota = lax.iota(jnp.int32, 16)

        def compute_chunk(q, buf):
            sb = lax.div(q, nbblk)
            b0 = lax.rem(q, nbblk) * bblk

            def pos_body(j, _):
                dl = sb * pblk + j
                lvec = jnp.full((16,), dl, jnp.int32)
                pes = [
                    pe_v[pl.ds(dl * D + c * LANES, 16)] for c in range(ncol)
                ]

                @plsc.parallel_loop(0, bblk, step=16)
                def _(g0):
                    toks16 = tok_v[dl, pl.ds(b0 + g0, 16)] * D

                    def row_loads(rr):
                        rb = jnp.broadcast_to(toks16[rr], (16,)) + iota
                        return [
                            plsc.load_gather(tab_v, [rb + (c * LANES)])
                            for c in range(ncol)
                        ]

                    def row_store(rr, gs):
                        for c in range(ncol):
                            buf[g0 + rr, j, pl.ds(c * LANES, 16)] = (
                                gs[c] + pes[c]
                            )

                    staged = row_loads(0)
                    for rr in range(1, 16):
                        nxt = row_loads(rr)
                        row_store(rr - 1, staged)
                        staged = nxt
                    row_store(15, staged)

                return 0

            lax.fori_loop(0, pblk, pos_body, 0)

        def out_slice(q):
            sb = lax.div(q, nbblk)
            b0 = lax.rem(q, nbblk) * bblk
            return out_hbm.at[pl.ds(b0, bblk), pl.ds(l0 + sb * pblk, pblk)]

        compute_chunk(0, buf0)
        pltpu.async_copy(buf0, out_slice(0), sem0)
        compute_chunk(1, buf1)
        pltpu.async_copy(buf1, out_slice(1), sem1)

        def pair(kk, _):
            q0 = kk * 2
            q1 = q0 + 1
            pltpu.make_async_copy(buf0, out_slice(q0 - 2), sem0).wait()
            compute_chunk(q0, buf0)
            pltpu.async_copy(buf0, out_slice(q0), sem0)
            pltpu.make_async_copy(buf1, out_slice(q1 - 2), sem1).wait()
            compute_chunk(q1, buf1)
            pltpu.async_copy(buf1, out_slice(q1), sem1)
            return 0

        lax.fori_loop(1, nchunk // 2, pair, 0)
        pltpu.make_async_copy(buf0, out_slice(nchunk - 2), sem0).wait()
        pltpu.make_async_copy(buf1, out_slice(nchunk - 1), sem1).wait()

    return sc_gather


def kernel(x, matbert_table, W, b):
    batch, seq = x.shape
    reduced = _reduced_table(matbert_table, W, b)
    pe = jnp.asarray(_PE_CONST)
    sc = _make_sc_kernel(batch)
    return sc(x.T, reduced.reshape(-1), pe)


# speedup vs baseline: 10.3744x; 1.0078x over previous
import functools

import numpy as np
import jax
import jax.numpy as jnp
from jax import lax
from jax.experimental import pallas as pl
from jax.experimental.pallas import tpu as pltpu
from jax.experimental.pallas import tpu_sc as plsc

VOCAB = 64
SEQ = 512
D = 128
H = 768

NUM_CORES = 2
NUM_SUBCORES = 16
NW = NUM_CORES * NUM_SUBCORES

LANES = 16
LPW = SEQ // NW


def _pe_flat():
    pos = np.arange(SEQ)[:, None].astype(np.float32)
    i = np.arange(D // 2)[None, :].astype(np.float32)
    ang = pos / np.power(10000.0, (2.0 * i) / float(D))
    pe = np.zeros((SEQ, D), dtype=np.float32)
    pe[:, 0::2] = np.sin(ang)
    pe[:, 1::2] = np.cos(ang)
    return pe.reshape(-1)


_PE_CONST = _pe_flat()


def _matmul_body(a_ref, w_ref, b_ref, o_ref):
    o_ref[...] = (
        jnp.dot(a_ref[...], w_ref[...], preferred_element_type=jnp.float32)
        + b_ref[...]
    )


def _reduced_table(matbert_table, W, b):
    return pl.pallas_call(
        _matmul_body,
        out_shape=jax.ShapeDtypeStruct((VOCAB, D), jnp.float32),
    )(matbert_table, W, b.reshape(1, D))


def _make_sc_kernel(batch):
    bblk = 16
    pblk = LPW
    nbblk = batch // bblk
    npblk = LPW // pblk
    nchunk = npblk * nbblk
    ncol = D // LANES

    mesh = plsc.VectorSubcoreMesh(
        core_axis_name="c",
        subcore_axis_name="s",
        num_cores=NUM_CORES,
        num_subcores=NUM_SUBCORES,
    )

    @functools.partial(
        pl.kernel,
        out_type=jax.ShapeDtypeStruct((batch * SEQ * D,), jnp.float32),
        mesh=mesh,
        compiler_params=pltpu.CompilerParams(needs_layout_passes=False),
        scratch_types=[
            pltpu.VMEM((VOCAB * D,), jnp.float32),
            pltpu.VMEM((LPW * D,), jnp.float32),
            pltpu.VMEM((LPW, batch), jnp.int32),
            pltpu.VMEM((bblk * pblk * D,), jnp.float32),
            pltpu.VMEM((bblk * pblk * D,), jnp.float32),
            pltpu.SemaphoreType.DMA,
            pltpu.SemaphoreType.DMA,
        ],
    )
    def sc_gather(
        xt_hbm, red_hbm, pe_hbm, out_hbm, tab_v, pe_v, tok_v, buf0, buf1,
        sem0, sem1,
    ):
        wid = lax.axis_index("s") * NUM_CORES + lax.axis_index("c")
        l0 = wid * LPW
        pltpu.sync_copy(red_hbm, tab_v)
        pltpu.sync_copy(pe_hbm.at[pl.ds(l0 * D, LPW * D)], pe_v)
        pltpu.sync_copy(xt_hbm.at[pl.ds(l0, LPW)], tok_v)

        iota = lax.iota(jnp.int32, 16)

        def compute_chunk(q, buf):
            sb = lax.div(q, nbblk)
            b0 = lax.rem(q, nbblk) * bblk

            def pos_body(j, _):
                dl = sb * pblk + j
                lvec = jnp.full((16,), dl, jnp.int32)
                pes = [
                    pe_v[pl.ds(dl * D + c * LANES, 16)] for c in range(ncol)
                ]

                @plsc.parallel_loop(0, bblk, step=16)
                def _(g0):
                    toks16 = tok_v[dl, pl.ds(b0 + g0, 16)] * D

                    def row_loads(rr):
                        rb = jnp.broadcast_to(toks16[rr], (16,)) + iota
                        return [
                            plsc.load_gather(tab_v, [rb + (c * LANES)])
                            for c in range(ncol)
                        ]

                    def row_store(rr, gs):
                        base = ((g0 + rr) * pblk + j) * D
                        for c in range(ncol):
                            buf[pl.ds(base + c * LANES, 16)] = gs[c] + pes[c]

                    staged = row_loads(0)
                    for rr in range(1, 16):
                        nxt = row_loads(rr)
                        row_store(rr - 1, staged)
                        staged = nxt
                    row_store(15, staged)

                return 0

            lax.fori_loop(0, pblk, pos_body, 0)

        def start_out(q, buf, sem):
            sb = lax.div(q, nbblk)
            b0 = lax.rem(q, nbblk) * bblk
            l8 = l0 + sb * pblk
            for i in range(bblk):
                pltpu.async_copy(
                    buf.at[pl.ds(i * pblk * D, pblk * D)],
                    out_hbm.at[pl.ds(((b0 + i) * SEQ + l8) * D, pblk * D)],
                    sem,
                )

        def drain_out(buf, sem):
            pltpu.make_async_copy(
                buf, out_hbm.at[pl.ds(0, bblk * pblk * D)], sem
            ).wait()

        compute_chunk(0, buf0)
        start_out(0, buf0, sem0)
        compute_chunk(1, buf1)
        start_out(1, buf1, sem1)

        def pair(kk, _):
            q0 = kk * 2
            q1 = q0 + 1
            drain_out(buf0, sem0)
            compute_chunk(q0, buf0)
            start_out(q0, buf0, sem0)
            drain_out(buf1, sem1)
            compute_chunk(q1, buf1)
            start_out(q1, buf1, sem1)
            return 0

        lax.fori_loop(1, nchunk // 2, pair, 0)
        drain_out(buf0, sem0)
        drain_out(buf1, sem1)

    return sc_gather


def kernel(x, matbert_table, W, b):
    batch, seq = x.shape
    reduced = _reduced_table(matbert_table, W, b)
    pe = jnp.asarray(_PE_CONST)
    sc = _make_sc_kernel(batch)
    return sc(x.T, reduced.reshape(-1), pe).reshape(batch, seq, D)


# speedup vs baseline: 10.3869x; 1.0012x over previous
import functools

import numpy as np
import jax
import jax.numpy as jnp
from jax import lax
from jax.experimental import pallas as pl
from jax.experimental.pallas import tpu as pltpu
from jax.experimental.pallas import tpu_sc as plsc

VOCAB = 64
SEQ = 512
D = 128
H = 768

NUM_CORES = 2
NUM_SUBCORES = 16
NW = NUM_CORES * NUM_SUBCORES

LANES = 16
LPW = SEQ // NW


def _pe_flat():
    pos = np.arange(SEQ)[:, None].astype(np.float32)
    i = np.arange(D // 2)[None, :].astype(np.float32)
    ang = pos / np.power(10000.0, (2.0 * i) / float(D))
    pe = np.zeros((SEQ, D), dtype=np.float32)
    pe[:, 0::2] = np.sin(ang)
    pe[:, 1::2] = np.cos(ang)
    return pe.reshape(-1)


_PE_CONST = _pe_flat()


def _matmul_body(a_ref, w_ref, b_ref, o_ref):
    o_ref[...] = (
        jnp.dot(a_ref[...], w_ref[...], preferred_element_type=jnp.float32)
        + b_ref[...]
    )


def _reduced_table(matbert_table, W, b):
    return pl.pallas_call(
        _matmul_body,
        out_shape=jax.ShapeDtypeStruct((VOCAB, D), jnp.float32),
    )(matbert_table, W, b.reshape(1, D))


def _make_sc_kernel(batch):
    bblk = 16
    pblk = LPW
    nbblk = batch // bblk
    npblk = LPW // pblk
    nchunk = npblk * nbblk
    ncol = D // LANES

    mesh = plsc.VectorSubcoreMesh(
        core_axis_name="c",
        subcore_axis_name="s",
        num_cores=NUM_CORES,
        num_subcores=NUM_SUBCORES,
    )

    @functools.partial(
        pl.kernel,
        out_type=jax.ShapeDtypeStruct((batch * SEQ * D,), jnp.float32),
        mesh=mesh,
        compiler_params=pltpu.CompilerParams(needs_layout_passes=False),
        scratch_types=[
            pltpu.VMEM((VOCAB * D,), jnp.float32),
            pltpu.VMEM((LPW * D,), jnp.float32),
            pltpu.VMEM((LPW, batch), jnp.int32),
            pltpu.VMEM((bblk * pblk * D,), jnp.float32),
            pltpu.VMEM((bblk * pblk * D,), jnp.float32),
            pltpu.SemaphoreType.DMA,
            pltpu.SemaphoreType.DMA,
        ],
    )
    def sc_gather(
        xt_hbm, red_hbm, pe_hbm, out_hbm, tab_v, pe_v, tok_v, buf0, buf1,
        sem0, sem1,
    ):
        wid = lax.axis_index("s") * NUM_CORES + lax.axis_index("c")
        l0 = wid * LPW
        pltpu.sync_copy(red_hbm, tab_v)
        pltpu.sync_copy(pe_hbm.at[pl.ds(l0 * D, LPW * D)], pe_v)
        pltpu.sync_copy(xt_hbm.at[pl.ds(l0, LPW)], tok_v)

        iota = lax.iota(jnp.int32, 16)

        def compute_chunk(q, buf):
            sb = lax.div(q, nbblk)
            b0 = lax.rem(q, nbblk) * bblk

            def pos_body(j, _):
                dl = sb * pblk + j
                lvec = jnp.full((16,), dl, jnp.int32)
                pes = [
                    pe_v[pl.ds(dl * D + c * LANES, 16)] for c in range(ncol)
                ]

                @plsc.parallel_loop(0, bblk, step=16)
                def _(g0):
                    toks16 = tok_v[dl, pl.ds(b0 + g0, 16)] * D

                    def row_loads(rr):
                        rb = jnp.broadcast_to(toks16[rr], (16,)) + iota
                        return [
                            plsc.load_gather(
                                tab_v.at[pl.ds(c * LANES, VOCAB * D - c * LANES)],
                                [rb],
                            )
                            for c in range(ncol)
                        ]

                    def row_store(rr, gs):
                        base = ((g0 + rr) * pblk + j) * D
                        for c in range(ncol):
                            buf[pl.ds(base + c * LANES, 16)] = gs[c] + pes[c]

                    staged = row_loads(0)
                    for rr in range(1, 16):
                        nxt = row_loads(rr)
                        row_store(rr - 1, staged)
                        staged = nxt
                    row_store(15, staged)

                return 0

            lax.fori_loop(0, pblk, pos_body, 0)

        def start_out(q, buf, sem):
            sb = lax.div(q, nbblk)
            b0 = lax.rem(q, nbblk) * bblk
            l8 = l0 + sb * pblk
            for i in range(bblk):
                pltpu.async_copy(
                    buf.at[pl.ds(i * pblk * D, pblk * D)],
                    out_hbm.at[pl.ds(((b0 + i) * SEQ + l8) * D, pblk * D)],
                    sem,
                )

        def drain_out(buf, sem):
            pltpu.make_async_copy(
                buf, out_hbm.at[pl.ds(0, bblk * pblk * D)], sem
            ).wait()

        compute_chunk(0, buf0)
        start_out(0, buf0, sem0)
        compute_chunk(1, buf1)
        start_out(1, buf1, sem1)

        def pair(kk, _):
            q0 = kk * 2
            q1 = q0 + 1
            drain_out(buf0, sem0)
            compute_chunk(q0, buf0)
            start_out(q0, buf0, sem0)
            drain_out(buf1, sem1)
            compute_chunk(q1, buf1)
            start_out(q1, buf1, sem1)
            return 0

        lax.fori_loop(1, nchunk // 2, pair, 0)
        drain_out(buf0, sem0)
        drain_out(buf1, sem1)

    return sc_gather


def kernel(x, matbert_table, W, b):
    batch, seq = x.shape
    reduced = _reduced_table(matbert_table, W, b)
    pe = jnp.asarray(_PE_CONST)
    sc = _make_sc_kernel(batch)
    return sc(x.T, reduced.reshape(-1), pe).reshape(batch, seq, D)


# speedup vs baseline: 15.4129x; 1.4839x over previous
import functools

import numpy as np
import jax
import jax.numpy as jnp
from jax import lax
from jax.experimental import pallas as pl
from jax.experimental.pallas import tpu as pltpu
from jax.experimental.pallas import tpu_sc as plsc

VOCAB = 64
SEQ = 512
D = 128
H = 768

NUM_CORES = 2
NUM_SUBCORES = 16
NW = NUM_CORES * NUM_SUBCORES

LANES = 16
LPW = SEQ // NW


def _pe_flat():
    pos = np.arange(SEQ)[:, None].astype(np.float32)
    i = np.arange(D // 2)[None, :].astype(np.float32)
    ang = pos / np.power(10000.0, (2.0 * i) / float(D))
    pe = np.zeros((SEQ, D), dtype=np.float32)
    pe[:, 0::2] = np.sin(ang)
    pe[:, 1::2] = np.cos(ang)
    return pe.reshape(-1)


_PE_CONST = _pe_flat()


def _matmul_body(a_ref, w_ref, b_ref, o_ref):
    o_ref[...] = (
        jnp.dot(a_ref[...], w_ref[...], preferred_element_type=jnp.float32)
        + b_ref[...]
    )


def _reduced_table(matbert_table, W, b):
    return pl.pallas_call(
        _matmul_body,
        out_shape=jax.ShapeDtypeStruct((VOCAB, D), jnp.float32),
    )(matbert_table, W, b.reshape(1, D))


def _make_sc_kernel(batch):
    bblk = 16
    pblk = LPW
    nbblk = batch // bblk
    npblk = LPW // pblk
    nchunk = npblk * nbblk
    ncol = D // LANES

    mesh = plsc.VectorSubcoreMesh(
        core_axis_name="c",
        subcore_axis_name="s",
        num_cores=NUM_CORES,
        num_subcores=NUM_SUBCORES,
    )

    @functools.partial(
        pl.kernel,
        out_type=jax.ShapeDtypeStruct((batch * SEQ * D,), jnp.float32),
        mesh=mesh,
        compiler_params=pltpu.CompilerParams(needs_layout_passes=False),
        scratch_types=[
            pltpu.VMEM((VOCAB * D,), jnp.float32),
            pltpu.VMEM((LPW * D,), jnp.float32),
            pltpu.VMEM((LPW, batch), jnp.int32),
            pltpu.VMEM((bblk * pblk * D,), jnp.float32),
            pltpu.VMEM((bblk * pblk * D,), jnp.float32),
            pltpu.SemaphoreType.DMA,
            pltpu.SemaphoreType.DMA,
        ],
    )
    def sc_gather(
        xt_hbm, red_hbm, pe_hbm, out_hbm, tab_v, pe_v, tok_v, buf0, buf1,
        sem0, sem1,
    ):
        wid = lax.axis_index("s") * NUM_CORES + lax.axis_index("c")
        l0 = wid * LPW
        pltpu.sync_copy(red_hbm, tab_v)
        pltpu.sync_copy(pe_hbm.at[pl.ds(l0 * D, LPW * D)], pe_v)
        pltpu.sync_copy(xt_hbm.at[pl.ds(l0, LPW)], tok_v)

        iota = lax.iota(jnp.int32, 16)

        def compute_chunk(q, buf):
            sb = lax.div(q, nbblk)
            b0 = lax.rem(q, nbblk) * bblk

            def pos_body(j, _):
                dl = sb * pblk + j
                lvec = jnp.full((16,), dl, jnp.int32)
                pes = [
                    pe_v[pl.ds(dl * D + c * LANES, 16)] for c in range(ncol)
                ]

                @plsc.parallel_loop(0, bblk, step=16)
                def _(g0):
                    toks16 = tok_v[dl, pl.ds(b0 + g0, 16)] * D

                    def row_loads(rr):
                        rb = jnp.broadcast_to(toks16[rr], (16,)) + iota
                        return [
                            plsc.load_gather(
                                tab_v.at[pl.ds(c * LANES, VOCAB * D - c * LANES)],
                                [rb],
                            )
                            for c in range(ncol)
                        ]

                    def st(rr, c, val):
                        base = ((g0 + rr) * pblk + j) * D
                        buf[pl.ds(base + c * LANES, 16)] = val + pes[c]

                    staged = row_loads(0)
                    for rr in range(1, 16):
                        rb = jnp.broadcast_to(toks16[rr], (16,)) + iota
                        nxt = []
                        for c in range(ncol):
                            nxt.append(plsc.load_gather(
                                tab_v.at[pl.ds(c * LANES, VOCAB * D - c * LANES)],
                                [rb],
                            ))
                            st(rr - 1, c, staged[c])
                        staged = nxt
                    for c in range(ncol):
                        st(15, c, staged[c])

                return 0

            lax.fori_loop(0, pblk, pos_body, 0)

        def start_out(q, buf, sem):
            sb = lax.div(q, nbblk)
            b0 = lax.rem(q, nbblk) * bblk
            l8 = l0 + sb * pblk
            for i in range(bblk):
                pltpu.async_copy(
                    buf.at[pl.ds(i * pblk * D, pblk * D)],
                    out_hbm.at[pl.ds(((b0 + i) * SEQ + l8) * D, pblk * D)],
                    sem,
                )

        def drain_out(buf, sem):
            pltpu.make_async_copy(
                buf, out_hbm.at[pl.ds(0, bblk * pblk * D)], sem
            ).wait()

        compute_chunk(0, buf0)
        start_out(0, buf0, sem0)
        compute_chunk(1, buf1)
        start_out(1, buf1, sem1)

        def pair(kk, _):
            q0 = kk * 2
            q1 = q0 + 1
            drain_out(buf0, sem0)
            compute_chunk(q0, buf0)
            start_out(q0, buf0, sem0)
            drain_out(buf1, sem1)
            compute_chunk(q1, buf1)
            start_out(q1, buf1, sem1)
            return 0

        lax.fori_loop(1, nchunk // 2, pair, 0)
        drain_out(buf0, sem0)
        drain_out(buf1, sem1)

    return sc_gather


def kernel(x, matbert_table, W, b):
    batch, seq = x.shape
    reduced = _reduced_table(matbert_table, W, b)
    pe = jnp.asarray(_PE_CONST)
    sc = _make_sc_kernel(batch)
    return sc(x.T, reduced.reshape(-1), pe).reshape(batch, seq, D)
